# Initial kernel scaffold; baseline (speedup 1.0000x reference)
import jax
import jax.numpy as jnp
from jax.experimental import pallas as pl


def kernel(pred, target):
    raise NotImplementedError("write your pallas kernel here")


# speedup vs baseline: 1294.8125x; 1294.8125x over previous
import functools

import jax
import jax.numpy as jnp
from jax import lax
from jax.experimental import pallas as pl
from jax.experimental.pallas import tpu as pltpu
from jax.experimental.pallas import tpu_sc as plsc

N = 8192
L = 16
NT = 16
CHUNK = N // NT
BPT = CHUNK // L
NEG = -3.0e38


def _f32(x):
    return x.astype(jnp.float32) if hasattr(x, "astype") else jnp.float32(x)


def _sget(ref, i):
    return ref[pl.ds(i, L)][0]


def _sset(ref, i, v):
    plsc.store_scatter(ref, [jnp.full((L,), i, jnp.int32)],
                       jnp.full((L,), v), mask=lax.iota(jnp.int32, L) == 0)


def _sc_body(vals_hbm, out_hbm, ck, cp, xk, xp, psum, pcnt, asum, acnt,
             gsum, gcnt, skey, spay, fb, pout, cbuf,
             sh_k, sh_p, sh_sum, sh_cnt, sh_n):
    c = lax.axis_index("c")
    t = lax.axis_index("s")
    base = t * CHUNK

    pltpu.sync_copy(vals_hbm.at[pl.ds(c * N + base, CHUNK)],
                    ck.at[pl.ds(0, CHUNK)])

    def init_blk(bi, _):
        cp[pl.ds(bi * L, L)] = lax.iota(jnp.int32, L) + (base + bi * L)
        return 0
    lax.fori_loop(0, BPT, init_blk, 0)

    def vsort_blk(bi, desc):
        off = bi * L
        sk, sv = plsc.sort_key_val(ck[pl.ds(off, L)], cp[pl.ds(off, L)],
                                   descending=desc)
        ck[pl.ds(off, L)] = sk
        cp[pl.ds(off, L)] = sv

    for bi in range(BPT):
        vsort_blk(bi, desc=(bi % 2 == 1))

    def exchange_step(J, K):
        s = J.bit_length() - 1

        def body(rr, _):
            low = rr & (J - 1)
            r = ((rr >> s) << (s + 1)) | low
            up_i = (t * BPT + r) & K
            upv = jnp.full((L,), up_i) == 0
            o1 = r * L
            o2 = (r + J) * L
            A = ck[pl.ds(o1, L)]
            VA = cp[pl.ds(o1, L)]
            B = ck[pl.ds(o2, L)]
            VB = cp[pl.ds(o2, L)]
            le = A <= B
            cm = (le & upv) | (~le & ~upv)
            ck[pl.ds(o1, L)] = jnp.where(cm, A, B)
            cp[pl.ds(o1, L)] = jnp.where(cm, VA, VB)
            ck[pl.ds(o2, L)] = jnp.where(cm, B, A)
            cp[pl.ds(o2, L)] = jnp.where(cm, VB, VA)
            return 0
        lax.fori_loop(0, BPT // 2, body, 0)

    def cross_step(JT, KT):
        peer = t ^ JT
        upv = jnp.full((L,), t & KT) == 0
        lov = jnp.full((L,), t & JT) == 0
        pltpu.sync_copy(ck.at[pl.ds(0, CHUNK)], sh_k.at[pl.ds(base, CHUNK)])
        pltpu.sync_copy(cp, sh_p.at[pl.ds(base, CHUNK)])
        plsc.subcore_barrier()
        pltpu.sync_copy(sh_k.at[pl.ds(peer * CHUNK, CHUNK)], xk)
        pltpu.sync_copy(sh_p.at[pl.ds(peer * CHUNK, CHUNK)], xp)

        def body(bi, _):
            off = bi * L
            M = ck[pl.ds(off, L)]
            VM = cp[pl.ds(off, L)]
            P = xk[pl.ds(off, L)]
            VP = xp[pl.ds(off, L)]
            c1 = M <= P
            c2 = P <= M
            side = (c1 & lov) | (c2 & ~lov)
            cond = (side & upv) | (~side & ~upv)
            ck[pl.ds(off, L)] = jnp.where(cond, M, P)
            cp[pl.ds(off, L)] = jnp.where(cond, VM, VP)
            return 0
        lax.fori_loop(0, BPT, body, 0)
        plsc.subcore_barrier()

    def vsort_pass(K):
        if K <= L:
            for bi in range(BPT):
                vsort_blk(bi, desc=((bi & K) != 0))
        else:
            KT = K // BPT
            flag = (t & KT) == 0

            @pl.when(flag)
            def _():
                def b_asc(bi, _):
                    vsort_blk(bi, desc=False)
                    return 0
                lax.fori_loop(0, BPT, b_asc, 0)

            @pl.when(jnp.logical_not(flag))
            def _():
                def b_desc(bi, _):
                    vsort_blk(bi, desc=True)
                    return 0
                lax.fori_loop(0, BPT, b_desc, 0)

    k = 2 * L
    while k <= N:
        K = k // L
        j = k // 2
        while j >= L:
            J = j // L
            if J < BPT:
                exchange_step(J, K)
            else:
                cross_step(J // BPT, K // BPT)
            j //= 2
        vsort_pass(K)
        k *= 2

    pltpu.sync_copy(ck.at[pl.ds(0, CHUNK)], sh_k.at[pl.ds(base, CHUNK)])
    pltpu.sync_copy(cp, sh_p.at[pl.ds(base, CHUNK)])

    def pav_body(q, carry):
        top, ts, tc = carry
        z = _sget(ck, q) - _f32(base + q + 1)

        def cond(st):
            top_, ts_, tc_, cs_, cc_ = st
            return (top_ >= 0) & (ts_ * cc_ > cs_ * tc_)

        def mbody(st):
            top_, ts_, tc_, cs_, cc_ = st
            cs2 = cs_ + ts_
            cc2 = cc_ + tc_
            top2 = top_ - 1
            i = jnp.maximum(top2, 0)
            return (top2, _sget(psum, i), _sget(pcnt, i), cs2, cc2)

        top, ts, tc, cs, cc = lax.while_loop(
            cond, mbody, (top, ts, tc, z, jnp.float32(1.0)))
        top = top + 1
        _sset(psum, top, cs)
        _sset(pcnt, top, cc)
        return (top, cs, cc)

    top, _, _ = lax.fori_loop(
        0, CHUNK, pav_body, (jnp.int32(-1), jnp.float32(0.0), jnp.float32(0.0)))

    pltpu.sync_copy(psum.at[pl.ds(0, CHUNK)], sh_sum.at[pl.ds(base, CHUNK)])
    pltpu.sync_copy(pcnt.at[pl.ds(0, CHUNK)], sh_cnt.at[pl.ds(base, CHUNK)])
    cbuf[...] = jnp.full((L,), top + 1, jnp.int32)
    pltpu.sync_copy(cbuf, sh_n.at[pl.ds(t * L, L)])
    plsc.subcore_barrier()

    @pl.when(t == 0)
    def _():
        pltpu.sync_copy(sh_k, skey)
        pltpu.sync_copy(sh_p, spay)

        def per_tile(tt, carry):
            pltpu.sync_copy(sh_sum.at[pl.ds(tt * CHUNK, CHUNK)],
                            asum.at[pl.ds(0, CHUNK)])
            pltpu.sync_copy(sh_cnt.at[pl.ds(tt * CHUNK, CHUNK)],
                            acnt.at[pl.ds(0, CHUNK)])
            pltpu.sync_copy(sh_n.at[pl.ds(tt * L, L)], cbuf)
            nat = cbuf[...][0]

            def atom(qq, carry2):
                gtop, ts, tc = carry2
                cs = _sget(asum, qq)
                cc = _sget(acnt, qq)

                def cond(st):
                    gtop_, ts_, tc_, cs_, cc_ = st
                    return (gtop_ >= 0) & (ts_ * cc_ > cs_ * tc_)

                def mbody(st):
                    gtop_, ts_, tc_, cs_, cc_ = st
                    cs2 = cs_ + ts_
                    cc2 = cc_ + tc_
                    gtop2 = gtop_ - 1
                    i = jnp.maximum(gtop2, 0)
                    return (gtop2, _sget(gsum, i), _sget(gcnt, i), cs2, cc2)

                gtop, ts, tc, cs, cc = lax.while_loop(
                    cond, mbody, carry2 + (cs, cc))
                gtop = gtop + 1
                _sset(gsum, gtop, cs)
                _sset(gcnt, gtop, cc)
                return (gtop, cs, cc)

            return lax.fori_loop(0, nat, atom, carry)

        gtop, _, _ = lax.fori_loop(
            0, NT, per_tile,
            (jnp.int32(-1), jnp.float32(0.0), jnp.float32(0.0)))
        npools = gtop + 1

        def initf(b, _):
            fb[pl.ds(b * L, L)] = jnp.full((L,), NEG, jnp.float32)
            return 0
        lax.fori_loop(0, N // L, initf, 0)

        def place(p, start):
            gc = _sget(gcnt, p)
            sv = jnp.full((L,), _sget(gsum, p))
            cv = jnp.full((L,), gc)
            _sset(fb, start, jnp.max(sv / cv))
            return start + gc.astype(jnp.int32)
        lax.fori_loop(0, npools, place, jnp.int32(0))

        def cmx(b, m):
            v = fb[pl.ds(b * L, L)]
            r = jnp.maximum(plsc.cummax(v), jnp.full((L,), m))
            fb[pl.ds(b * L, L)] = r
            return jnp.max(r)
        lax.fori_loop(0, N // L, cmx, jnp.float32(NEG))

        def scat(b, _):
            off = b * L
            pk = skey[pl.ds(off, L)] - fb[pl.ds(off, L)]
            plsc.store_scatter(pout, [spay[pl.ds(off, L)]], pk)
            return 0
        lax.fori_loop(0, N // L, scat, 0)

        pltpu.sync_copy(pout, out_hbm.at[pl.ds(c * N, N)])


_sc_mesh = plsc.VectorSubcoreMesh(
    core_axis_name="c", subcore_axis_name="s", num_cores=2, num_subcores=16)

_sc_soft_rank = functools.partial(
    pl.kernel,
    out_type=jax.ShapeDtypeStruct((2 * N,), jnp.float32),
    mesh=_sc_mesh,
    compiler_params=pltpu.CompilerParams(needs_layout_passes=False),
    scratch_types=[
        pltpu.VMEM((CHUNK + L,), jnp.float32),
        pltpu.VMEM((CHUNK,), jnp.int32),
        pltpu.VMEM((CHUNK,), jnp.float32),
        pltpu.VMEM((CHUNK,), jnp.int32),
        pltpu.VMEM((CHUNK + L,), jnp.float32),
        pltpu.VMEM((CHUNK + L,), jnp.float32),
        pltpu.VMEM((CHUNK + L,), jnp.float32),
        pltpu.VMEM((CHUNK + L,), jnp.float32),
        pltpu.VMEM((N + L,), jnp.float32),
        pltpu.VMEM((N + L,), jnp.float32),
        pltpu.VMEM((N,), jnp.float32),
        pltpu.VMEM((N,), jnp.int32),
        pltpu.VMEM((N,), jnp.float32),
        pltpu.VMEM((N,), jnp.float32),
        pltpu.VMEM((L,), jnp.int32),
        pltpu.VMEM_SHARED((N,), jnp.float32),
        pltpu.VMEM_SHARED((N,), jnp.int32),
        pltpu.VMEM_SHARED((N,), jnp.float32),
        pltpu.VMEM_SHARED((N,), jnp.float32),
        pltpu.VMEM_SHARED((NT * L,), jnp.int32),
    ],
)(_sc_body)


def _loss_body(r_ref, o_ref):
    x = r_ref[...]
    m = jnp.sum(x, axis=1, keepdims=True) * (1.0 / N)
    xc = x - m
    ss = jnp.sum(xc * xc, axis=1, keepdims=True)
    xn = xc * lax.rsqrt(ss)
    o_ref[0, 0] = 1.0 - jnp.sum(xn[0:1, :] * xn[1:2, :])


_loss = pl.pallas_call(
    _loss_body,
    out_shape=jax.ShapeDtypeStruct((1, 1), jnp.float32),
    in_specs=[pl.BlockSpec(memory_space=pltpu.VMEM)],
    out_specs=pl.BlockSpec(memory_space=pltpu.SMEM),
)


def kernel(pred, target):
    vals = jnp.concatenate([pred.reshape(-1), target.reshape(-1)], axis=0)
    ranks = _sc_soft_rank(vals)
    return _loss(ranks.reshape(2, N))[0, 0]
